# baseline (speedup 1.0000x reference)
import functools

import jax
import jax.numpy as jnp
from jax import lax
from jax.experimental import pallas as pl
from jax.experimental.pallas import tpu as pltpu
from jax.experimental.pallas import tpu_sc as plsc

_VOCAB = 100000
_D = 128
_B = 128
_S = 32
_NNEG = 20
_K = 2 + _NNEG
_T = _B * _S

_NC = 2
_NS = 16
_NW = _NC * _NS
_TPW = _T // _NW
_TC = 8
_NSTEP = _TPW // _TC
_ROWS = _TC * _K
_G = _ROWS // 2
_L = 16
_CHUNKS = _D // _L
_FL = 8
_SPS = _ROWS * _L
_PROWS = _T * _K * _L // _D


def _sc_gather_dot(W_o, idx2d, ivec):
    mesh = plsc.VectorSubcoreMesh(core_axis_name="c", subcore_axis_name="s")

    @functools.partial(
        pl.kernel,
        mesh=mesh,
        out_type=jax.ShapeDtypeStruct((_T * _K * _L // 2,), jnp.int32),
        scratch_types=[
            pltpu.VMEM((2 * _NSTEP, _G), jnp.int32),
            pltpu.VMEM((_TPW, _D), jnp.float32),
            pltpu.VMEM((2, _ROWS, _D), jnp.float32),
            pltpu.VMEM((_FL * _SPS,), jnp.int32),
            pltpu.SemaphoreType.DMA,
            pltpu.SemaphoreType.DMA,
        ],
    )
    def body(W_hbm, idx_hbm, ivec_hbm, out_hbm,
             idx_v, ivec_v, rows_v, out_v, sem_g, sem_o):
        wid = lax.axis_index("s") * _NC + lax.axis_index("c")

        pltpu.sync_copy(idx_hbm.at[pl.ds(wid * 2 * _NSTEP, 2 * _NSTEP)], idx_v)
        pltpu.sync_copy(ivec_hbm.at[pl.ds(wid * _TPW, _TPW)], ivec_v)

        def fire_gathers(s, b):
            c0 = pltpu.async_copy(W_hbm.at[idx_v.at[2 * s]],
                                  rows_v.at[b, pl.ds(0, _G)], sem_g)
            c1 = pltpu.async_copy(W_hbm.at[idx_v.at[2 * s + 1]],
                                  rows_v.at[b, pl.ds(_G, _G)], sem_g)
            return (c0, c1)

        def to_bf16_bits(x):
            u = lax.bitcast_convert_type(x, jnp.uint32)
            return (u + jnp.uint32(0x7FFF) + ((u >> 16) & jnp.uint32(1))) >> 16

        def compute(s):
            b = s % 2
            ob = (s // _FL) % 2
            sl = s % _FL

            def acc_of(r, ivs):
                acc = rows_v[b, r, pl.ds(0, _L)] * ivs[0]
                for c in range(1, _CHUNKS):
                    acc = acc + rows_v[b, r, pl.ds(c * _L, _L)] * ivs[c]
                return acc

            def t_body(t, _):
                ivs = [ivec_v[s * _TC + t, pl.ds(c * _L, _L)] for c in range(_CHUNKS)]

                def j_body(j2, _):
                    r = t * _K + 2 * j2
                    lo = to_bf16_bits(acc_of(r, ivs))
                    hi = to_bf16_bits(acc_of(r + 1, ivs))
                    pk = lax.bitcast_convert_type(lo | (hi << 16), jnp.int32)
                    out_v[pl.ds((ob * _FL + sl) * _SPS // 2 + r * _L // 2, _L)] = pk
                    return 0

                lax.fori_loop(0, _K // 2, j_body, 0)
                return 0

            lax.fori_loop(0, _TC, t_body, 0)

        n_group = _NSTEP // _FL
        gather_cp = {0: fire_gathers(0, 0)}
        out_cp = []
        for s in range(_NSTEP):
            g = s // _FL
            c0, c1 = gather_cp.pop(s)
            c0.wait()
            c1.wait()
            if s + 1 < _NSTEP:
                gather_cp[s + 1] = fire_gathers(s + 1, (s + 1) % 2)
            compute(s)
            if s % _FL == _FL - 1:
                half_sps = _FL * _SPS // 2
                off = pl.multiple_of(wid * (_NSTEP * _SPS // 2) + g * half_sps, 8)
                out_cp.append(pltpu.async_copy(
                    out_v.at[pl.ds((g % 2) * half_sps, half_sps)],
                    out_hbm.at[pl.ds(off, half_sps)], sem_o))
        for cp in out_cp:
            cp.wait()

    return body(W_o, idx2d, ivec)


def _tc_finish(partials2d):
    def body(p_ref, o_ref):
        x = p_ref[...].astype(jnp.float32)
        li = lax.broadcasted_iota(jnp.int32, (_D, 8), 0)
        ki = lax.broadcasted_iota(jnp.int32, (_D, 8), 1)
        gmat = ((li // 32 == ki // 2) & (li % 2 == ki % 2)).astype(jnp.float32)
        scores = jnp.dot(x, gmat, preferred_element_type=jnp.float32)
        ls = jnp.log(jax.nn.sigmoid(scores))
        dot_idx = (lax.broadcasted_iota(jnp.int32, (_PROWS, 8), 0) * 8
                   + lax.broadcasted_iota(jnp.int32, (_PROWS, 8), 1))
        w = jnp.where(dot_idx % _K < 2, 0.5, 1.0).astype(jnp.float32)
        o_ref[0, 0] = jnp.sum(ls * w) * (-1.0 / _T)

    out = pl.pallas_call(
        body,
        out_shape=jax.ShapeDtypeStruct((1, 1), jnp.float32),
        out_specs=pl.BlockSpec(memory_space=pltpu.SMEM),
    )(partials2d)
    return out[0, 0]


def kernel(iword_emb, owords, W_o):
    B, S, D = iword_emb.shape
    T = B * S

    nkey = jax.random.key(42)
    nwords = jax.random.randint(nkey, (T, _NNEG), 0, _VOCAB, dtype=jnp.int32)

    pos = jnp.arange(S)
    left = owords[:, jnp.maximum(pos - 1, 0)]
    right = owords[:, jnp.minimum(pos + 1, S - 1)]
    owin = jnp.stack([left, right], axis=-1).reshape(T, 2)

    idx2d = jnp.concatenate([owin, nwords], axis=1).reshape(_NW * 2 * _NSTEP, _G)
    ivec = iword_emb.reshape(T, D)

    bits = _sc_gather_dot(W_o, idx2d, ivec)
    p2d = lax.bitcast_convert_type(bits, jnp.bfloat16).reshape(_PROWS, _D)
    return _tc_finish(p2d)


# speedup vs baseline: 1.6951x; 1.6951x over previous
import functools

import jax
import jax.numpy as jnp
from jax import lax
from jax.experimental import pallas as pl
from jax.experimental.pallas import tpu as pltpu
from jax.experimental.pallas import tpu_sc as plsc

_VOCAB = 100000
_D = 128
_B = 128
_S = 32
_NNEG = 20
_K = 2 + _NNEG
_T = _B * _S

_NC = 2
_NS = 16
_NW = _NC * _NS
_TPW = _T // _NW
_TC = 8
_NSTEP = _TPW // _TC
_ROWS = _TC * _K
_G = _ROWS // 2
_L = 16
_CHUNKS = _D // _L
_FL = 8
_SPS = _ROWS * _L
_IROWS = _T * _K * _L // (2 * _D)


def _sc_gather_dot(W_o, idx2d, ivec):
    mesh = plsc.VectorSubcoreMesh(core_axis_name="c", subcore_axis_name="s")

    @functools.partial(
        pl.kernel,
        mesh=mesh,
        out_type=jax.ShapeDtypeStruct((_IROWS, _D), jnp.int32),
        scratch_types=[
            pltpu.VMEM((2 * _NSTEP, _G), jnp.int32),
            pltpu.VMEM((_TPW, _D), jnp.float32),
            pltpu.VMEM((2, _ROWS, _D), jnp.float32),
            pltpu.VMEM((2, _FL * 11, _D), jnp.int32),
            pltpu.SemaphoreType.DMA,
            pltpu.SemaphoreType.DMA,
        ],
    )
    def body(W_hbm, idx_hbm, ivec_hbm, out_hbm,
             idx_v, ivec_v, rows_v, out_v, sem_g, sem_o):
        wid = lax.axis_index("s") * _NC + lax.axis_index("c")

        pltpu.sync_copy(idx_hbm.at[pl.ds(wid * 2 * _NSTEP, 2 * _NSTEP)], idx_v)
        pltpu.sync_copy(ivec_hbm.at[pl.ds(wid * _TPW, _TPW)], ivec_v)

        def fire_gathers(s, b):
            c0 = pltpu.async_copy(W_hbm.at[idx_v.at[2 * s]],
                                  rows_v.at[b, pl.ds(0, _G)], sem_g)
            c1 = pltpu.async_copy(W_hbm.at[idx_v.at[2 * s + 1]],
                                  rows_v.at[b, pl.ds(_G, _G)], sem_g)
            return (c0, c1)

        def to_bf16_bits(x):
            u = lax.bitcast_convert_type(x, jnp.uint32)
            return (u + jnp.uint32(0x7FFF) + ((u >> 16) & jnp.uint32(1))) >> 16

        def compute(s):
            b = s % 2
            ob = (s // _FL) % 2
            sl = s % _FL

            def acc_of(r, ivs):
                acc = rows_v[b, r, pl.ds(0, _L)] * ivs[0]
                for c in range(1, _CHUNKS):
                    acc = acc + rows_v[b, r, pl.ds(c * _L, _L)] * ivs[c]
                return acc

            def t_body(t, _):
                ivs = [ivec_v[s * _TC + t, pl.ds(c * _L, _L)] for c in range(_CHUNKS)]

                def j_body(j2, _):
                    r = t * _K + 2 * j2
                    lo = to_bf16_bits(acc_of(r, ivs))
                    hi = to_bf16_bits(acc_of(r + 1, ivs))
                    pk = lax.bitcast_convert_type(lo | (hi << 16), jnp.int32)
                    p_ = t * (_K // 2) + j2
                    out_v[ob, sl * 11 + p_ // 8, pl.ds((p_ % 8) * _L, _L)] = pk
                    return 0

                lax.fori_loop(0, _K // 2, j_body, 0)
                return 0

            lax.fori_loop(0, _TC, t_body, 0)

        n_group = _NSTEP // _FL
        gather_cp = {0: fire_gathers(0, 0)}
        out_cp = []
        for s in range(_NSTEP):
            g = s // _FL
            c0, c1 = gather_cp.pop(s)
            c0.wait()
            c1.wait()
            if s + 1 < _NSTEP:
                gather_cp[s + 1] = fire_gathers(s + 1, (s + 1) % 2)
            compute(s)
            if s % _FL == _FL - 1:
                row0 = wid * _NSTEP * 11 + g * _FL * 11
                out_cp.append(pltpu.async_copy(
                    out_v.at[g % 2],
                    out_hbm.at[pl.ds(row0, _FL * 11)], sem_o))
        for cp in out_cp:
            cp.wait()

    return body(W_o, idx2d, ivec)


def _tc_finish(bits2d):
    def body(p_ref, o_ref):
        x = p_ref[...]
        f_lo = lax.bitcast_convert_type(x << 16, jnp.float32)
        f_hi = lax.bitcast_convert_type(x & jnp.int32(-65536), jnp.float32)
        li = lax.broadcasted_iota(jnp.int32, (_D, 8), 0)
        ki = lax.broadcasted_iota(jnp.int32, (_D, 8), 1)
        gmat = (li // _L == ki).astype(jnp.float32)
        s_lo = jnp.dot(f_lo, gmat, preferred_element_type=jnp.float32)
        s_hi = jnp.dot(f_hi, gmat, preferred_element_type=jnp.float32)
        ls = jnp.log(jax.nn.sigmoid(s_lo)) + jnp.log(jax.nn.sigmoid(s_hi))
        pair = (lax.broadcasted_iota(jnp.int32, (_IROWS, 8), 0) * 8
                + lax.broadcasted_iota(jnp.int32, (_IROWS, 8), 1))
        w = jnp.where(pair % (_K // 2) == 0, 0.5, 1.0).astype(jnp.float32)
        o_ref[0, 0] = jnp.sum(ls * w) * (-1.0 / _T)

    out = pl.pallas_call(
        body,
        out_shape=jax.ShapeDtypeStruct((1, 1), jnp.float32),
        out_specs=pl.BlockSpec(memory_space=pltpu.SMEM),
    )(bits2d)
    return out[0, 0]


def kernel(iword_emb, owords, W_o):
    B, S, D = iword_emb.shape
    T = B * S

    nkey = jax.random.key(42)
    nwords = jax.random.randint(nkey, (T, _NNEG), 0, _VOCAB, dtype=jnp.int32)

    pos = jnp.arange(S)
    left = owords[:, jnp.maximum(pos - 1, 0)]
    right = owords[:, jnp.minimum(pos + 1, S - 1)]
    owin = jnp.stack([left, right], axis=-1).reshape(T, 2)

    idx2d = jnp.concatenate([owin, nwords], axis=1).reshape(_NW * 2 * _NSTEP, _G)
    ivec = iword_emb.reshape(T, D)

    bits2d = _sc_gather_dot(W_o, idx2d, ivec)
    return _tc_finish(bits2d)
